# baseline (speedup 1.0000x reference)
import jax
import jax.numpy as jnp
from jax import lax
from jax.experimental import pallas as pl
from jax.experimental.pallas import tpu as pltpu
from jax.experimental.pallas import tpu_sc as plsc

FEATURE = 1000000
BATCH = 16384
N_FIELDS = 26
NUM_CORES = 2
NUM_SUBCORES = 16
NUM_WORKERS = NUM_CORES * NUM_SUBCORES
ROWS_PER_W = BATCH // NUM_WORKERS
IDX_PER_W = ROWS_PER_W * N_FIELDS
LANES = 16


def _prep_body(in_hbm, idx_hbm, raw_v, idx_v):
    wid = lax.axis_index("s") * NUM_CORES + lax.axis_index("c")
    base_r = wid * ROWS_PER_W

    pltpu.sync_copy(in_hbm.at[pl.ds(base_r, ROWS_PER_W), :], raw_v)
    lane = lax.iota(jnp.int32, LANES)

    @pl.loop(0, ROWS_PER_W // LANES)
    def _chunk(i):
        j = i * LANES + lane
        for f in range(N_FIELDS):
            v = plsc.load_gather(raw_v, [j, jnp.full((LANES,), f, jnp.int32)])
            idx_v[pl.ds(f * ROWS_PER_W + i * LANES, LANES)] = v

    pltpu.sync_copy(idx_v, idx_hbm.at[pl.ds(wid * IDX_PER_W, IDX_PER_W)])


def _sc_body(w_hbm, idx_hbm, out_hbm, idx_v, rows_v, out_v, sem):
    wid = lax.axis_index("s") * NUM_CORES + lax.axis_index("c")
    base_i = wid * IDX_PER_W
    base_o = wid * ROWS_PER_W

    pltpu.sync_copy(idx_hbm.at[pl.ds(base_i, IDX_PER_W)], idx_v)
    pltpu.async_copy(w_hbm.at[idx_v], rows_v, sem).wait()

    @pl.loop(0, ROWS_PER_W // LANES)
    def _chunk(i):
        b = i * LANES
        acc = rows_v[pl.ds(b, LANES)]
        for f in range(1, N_FIELDS):
            acc = acc + rows_v[pl.ds(f * ROWS_PER_W + b, LANES)]
        out_v[pl.ds(b, LANES)] = acc

    pltpu.sync_copy(out_v, out_hbm.at[pl.ds(base_o, ROWS_PER_W)])


@jax.jit
def kernel(inputs, w):
    mesh = plsc.VectorSubcoreMesh(core_axis_name="c", subcore_axis_name="s")
    idx_flat = pl.kernel(
        _prep_body,
        out_type=jax.ShapeDtypeStruct((BATCH * N_FIELDS,), jnp.int32),
        mesh=mesh,
        scratch_types=[
            pltpu.VMEM((ROWS_PER_W, N_FIELDS), jnp.int32),
            pltpu.VMEM((IDX_PER_W,), jnp.int32),
        ],
        compiler_params=pltpu.CompilerParams(
            use_tc_tiling_on_sc=False, needs_layout_passes=False
        ),
    )(inputs.astype(jnp.int32))
    out = pl.kernel(
        _sc_body,
        out_type=jax.ShapeDtypeStruct((BATCH,), jnp.float32),
        mesh=mesh,
        scratch_types=[
            pltpu.VMEM((IDX_PER_W,), jnp.int32),
            pltpu.VMEM((IDX_PER_W,), jnp.float32),
            pltpu.VMEM((ROWS_PER_W,), jnp.float32),
            pltpu.SemaphoreType.DMA,
        ],
    )(w.reshape(-1), idx_flat)
    return out.reshape(BATCH, 1)


# speedup vs baseline: 1.9766x; 1.9766x over previous
import jax
import jax.numpy as jnp
from jax import lax
from jax.experimental import pallas as pl
from jax.experimental.pallas import tpu as pltpu
from jax.experimental.pallas import tpu_sc as plsc

FEATURE = 1000000
FEATURE_PAD = 1 << 20
BATCH = 16384
N_FIELDS = 26
NUM_CORES = 2
NUM_SUBCORES = 16
NUM_WORKERS = NUM_CORES * NUM_SUBCORES
ROWS_PER_W = BATCH // NUM_WORKERS
IDX_PER_W = ROWS_PER_W * N_FIELDS
LANES = 16


def _sc_body(w_hbm, idx_hbm, out_hbm, idx_v, rows_v, out_v, sem):
    wid = lax.axis_index("s") * NUM_CORES + lax.axis_index("c")
    base_i = wid * IDX_PER_W
    base_o = wid * ROWS_PER_W

    pltpu.sync_copy(idx_hbm.at[pl.ds(base_i, IDX_PER_W)], idx_v)
    pltpu.async_copy(w_hbm.at[idx_v], rows_v, sem).wait()

    @pl.loop(0, ROWS_PER_W // LANES)
    def _chunk(i):
        b = i * LANES
        acc = rows_v[pl.ds(b, LANES)]
        for f in range(1, N_FIELDS):
            acc = acc + rows_v[pl.ds(f * ROWS_PER_W + b, LANES)]
        out_v[pl.ds(b, LANES)] = acc

    pltpu.sync_copy(out_v, out_hbm.at[pl.ds(base_o, ROWS_PER_W)])


@jax.jit
def kernel(inputs, w):
    idx_flat = (
        inputs.astype(jnp.int32)
        .reshape(NUM_WORKERS, ROWS_PER_W, N_FIELDS)
        .transpose(0, 2, 1)
        .reshape(-1)
    )
    w_flat = jnp.pad(w, ((0, FEATURE_PAD - FEATURE), (0, 0))).reshape(-1)
    mesh = plsc.VectorSubcoreMesh(core_axis_name="c", subcore_axis_name="s")
    out = pl.kernel(
        _sc_body,
        out_type=jax.ShapeDtypeStruct((BATCH,), jnp.float32),
        mesh=mesh,
        scratch_types=[
            pltpu.VMEM((IDX_PER_W,), jnp.int32),
            pltpu.VMEM((IDX_PER_W,), jnp.float32),
            pltpu.VMEM((ROWS_PER_W,), jnp.float32),
            pltpu.SemaphoreType.DMA,
        ],
    )(w_flat, idx_flat)
    return out.reshape(BATCH, 1)
